# baseline (speedup 1.0000x reference)
import functools
import math

import jax
import jax.numpy as jnp
from jax import lax
from jax.experimental import pallas as pl
from jax.experimental.pallas import tpu as pltpu
from jax.experimental.pallas import tpu_sc as plsc

_NC = 2
_NS = 16
_NW = _NC * _NS

N_HIDDEN = 256
N_EDGE_HIDDEN = 128
MSG_NORM = 100.0
_EPS = 1e-5


def _silu(x):
    return x * (1.0 / (1.0 + jnp.exp(-x)))


def _layernorm(x, g, b):
    m = jnp.mean(x, axis=-1, keepdims=True)
    v = jnp.mean((x - m) ** 2, axis=-1, keepdims=True)
    return (x - m) * jax.lax.rsqrt(v + _EPS) * g + b


def _row_block(bs, d):
    return pl.BlockSpec((bs, d), lambda i: (i, 0))


def _full_block(shape):
    nd = len(shape)
    return pl.BlockSpec(shape, lambda i: (0,) * nd)


def _node_embed_body(x_ref, r_ref, w1_ref, b1_ref, w2_ref, b2_ref,
                     lng_ref, lnb_ref, rw1_ref, rb1_ref, rw2_ref, rb2_ref,
                     gate_ref, o_ref):
    x = x_ref[...]
    h = _silu(jnp.dot(x, w1_ref[...]) + b1_ref[...])
    h = _silu(jnp.dot(h, w2_ref[...]) + b2_ref[...])
    h = _layernorm(h, lng_ref[...], lnb_ref[...])
    r = _silu(jnp.dot(r_ref[...], rw1_ref[...]) + rb1_ref[...])
    r = jnp.dot(r, rw2_ref[...]) + rb2_ref[...]
    o_ref[...] = h + gate_ref[0, 0] * r


def _node_embed(x, r, se, re, bs):
    n, din = x.shape
    dr = r.shape[1]
    d = N_HIDDEN
    return pl.pallas_call(
        _node_embed_body,
        grid=(n // bs,),
        in_specs=[
            _row_block(bs, din), _row_block(bs, dr),
            _full_block((din, d)), _full_block((1, d)),
            _full_block((d, d)), _full_block((1, d)),
            _full_block((1, d)), _full_block((1, d)),
            _full_block((dr, d)), _full_block((1, d)),
            _full_block((d, d)), _full_block((1, d)),
            _full_block((1, 1)),
        ],
        out_specs=_row_block(bs, d),
        out_shape=jax.ShapeDtypeStruct((n, d), jnp.float32),
    )(x, r,
      se["l1"]["W"], se["l1"]["b"].reshape(1, -1),
      se["l2"]["W"], se["l2"]["b"].reshape(1, -1),
      se["ln"]["g"].reshape(1, -1), se["ln"]["b"].reshape(1, -1),
      re["l1"]["W"], re["l1"]["b"].reshape(1, -1),
      re["l2"]["W"], re["l2"]["b"].reshape(1, -1),
      re["gate"].reshape(1, 1))


def _mlp_ln_body(x_ref, w1_ref, b1_ref, w2_ref, b2_ref, lng_ref, lnb_ref,
                 o_ref):
    h = _silu(jnp.dot(x_ref[...], w1_ref[...]) + b1_ref[...])
    h = _silu(jnp.dot(h, w2_ref[...]) + b2_ref[...])
    o_ref[...] = _layernorm(h, lng_ref[...], lnb_ref[...])


def _mlp_ln(x, p, bs):
    n, din = x.shape
    d = p["l1"]["W"].shape[1]
    return pl.pallas_call(
        _mlp_ln_body,
        grid=(n // bs,),
        in_specs=[
            _row_block(bs, din),
            _full_block((din, d)), _full_block((1, d)),
            _full_block((d, d)), _full_block((1, d)),
            _full_block((1, d)), _full_block((1, d)),
        ],
        out_specs=_row_block(bs, d),
        out_shape=jax.ShapeDtypeStruct((n, d), jnp.float32),
    )(x, p["l1"]["W"], p["l1"]["b"].reshape(1, -1),
      p["l2"]["W"], p["l2"]["b"].reshape(1, -1),
      p["ln"]["g"].reshape(1, -1), p["ln"]["b"].reshape(1, -1))


def _matmul_body(x_ref, w_ref, o_ref):
    o_ref[...] = jnp.dot(x_ref[...], w_ref[...]).astype(o_ref.dtype)


def _matmul(x, w, bs, out_dtype=jnp.float32):
    n, din = x.shape
    dout = w.shape[1]
    return pl.pallas_call(
        _matmul_body,
        grid=(n // bs,),
        in_specs=[_row_block(bs, din), _full_block((din, dout))],
        out_specs=_row_block(bs, dout),
        out_shape=jax.ShapeDtypeStruct((n, dout), out_dtype),
    )(x, w)


def _msg_body(hsw_ref, e_ref, w1b_ref, b1_ref, w2_ref, b2_ref, o_ref):
    h = jnp.dot(e_ref[...], w1b_ref[...])
    h = _silu(h + hsw_ref[...].astype(jnp.float32) + b1_ref[...])
    o_ref[...] = _silu(jnp.dot(h, w2_ref[...]) + b2_ref[...])


def _msg_mlp(hsw, e, cp, bs, e_off=0):
    n = hsw.shape[0]
    d = N_HIDDEN
    de = N_EDGE_HIDDEN
    ob = e_off // bs
    w1 = cp["msg1"]["W"]
    return pl.pallas_call(
        _msg_body,
        grid=(n // bs,),
        in_specs=[
            _row_block(bs, d),
            pl.BlockSpec((bs, de), lambda i: (i + ob, 0)),
            _full_block((de, d)), _full_block((1, d)),
            _full_block((d, d)), _full_block((1, d)),
        ],
        out_specs=_row_block(bs, d),
        out_shape=jax.ShapeDtypeStruct((n, d), jnp.float32),
    )(hsw, e, w1[d:], cp["msg1"]["b"].reshape(1, -1),
      cp["msg2"]["W"], cp["msg2"]["b"].reshape(1, -1))


def _node_update_body(h_ref, agg_ref, agg2_ref, ln1g_ref, ln1b_ref, w1_ref,
                      b1_ref, w2_ref, b2_ref, ln2g_ref, ln2b_ref, o_ref):
    agg = agg_ref[...] + agg2_ref[...]
    h = _layernorm(h_ref[...] + agg * (1.0 / MSG_NORM),
                   ln1g_ref[...], ln1b_ref[...])
    r = _silu(jnp.dot(h, w1_ref[...]) + b1_ref[...])
    r = _silu(jnp.dot(r, w2_ref[...]) + b2_ref[...])
    o_ref[...] = _layernorm(h + r, ln2g_ref[...], ln2b_ref[...])


def _node_update(h, agg, agg2, cp, bs):
    n = h.shape[0]
    d = N_HIDDEN
    return pl.pallas_call(
        _node_update_body,
        grid=(n // bs,),
        in_specs=[
            _row_block(bs, d), _row_block(bs, d), _row_block(bs, d),
            _full_block((1, d)), _full_block((1, d)),
            _full_block((d, d)), _full_block((1, d)),
            _full_block((d, d)), _full_block((1, d)),
            _full_block((1, d)), _full_block((1, d)),
        ],
        out_specs=_row_block(bs, d),
        out_shape=jax.ShapeDtypeStruct((n, d), jnp.float32),
    )(h, agg, agg2,
      cp["ln1"]["g"].reshape(1, -1), cp["ln1"]["b"].reshape(1, -1),
      cp["upd1"]["W"], cp["upd1"]["b"].reshape(1, -1),
      cp["upd2"]["W"], cp["upd2"]["b"].reshape(1, -1),
      cp["ln2"]["g"].reshape(1, -1), cp["ln2"]["b"].reshape(1, -1))


def _edge_update_body(ga_ref, gb_ref, e_ref, w1c_ref, b1_ref, w2_ref,
                      b2_ref, lng_ref, lnb_ref, o_ref):
    h = jnp.dot(e_ref[...], w1c_ref[...])
    h = _silu(h + ga_ref[...].astype(jnp.float32)
              + gb_ref[...].astype(jnp.float32) + b1_ref[...])
    eo = _silu(jnp.dot(h, w2_ref[...]) + b2_ref[...])
    o_ref[...] = _layernorm(e_ref[...] + eo, lng_ref[...], lnb_ref[...])


def _edge_update(ga, gb, e, ep, bs):
    n = ga.shape[0]
    de = N_EDGE_HIDDEN
    w1 = ep["l1"]["W"]
    d = N_HIDDEN
    return pl.pallas_call(
        _edge_update_body,
        grid=(n // bs,),
        in_specs=[
            _row_block(bs, de), _row_block(bs, de), _row_block(bs, de),
            _full_block((de, de)), _full_block((1, de)),
            _full_block((de, de)), _full_block((1, de)),
            _full_block((1, de)), _full_block((1, de)),
        ],
        out_specs=_row_block(bs, de),
        out_shape=jax.ShapeDtypeStruct((n, de), jnp.float32),
    )(ga, gb, e, w1[2 * d:],
      ep["l1"]["b"].reshape(1, -1),
      ep["l2"]["W"], ep["l2"]["b"].reshape(1, -1),
      ep["ln"]["g"].reshape(1, -1), ep["ln"]["b"].reshape(1, -1))


def _head_body(x_ref, w1_ref, b1_ref, w2_ref, b2_ref, o_ref):
    h = _silu(jnp.dot(x_ref[...], w1_ref[...]) + b1_ref[...])
    o_ref[...] = jnp.dot(h, w2_ref[...]) + b2_ref[...]


def _head(x, hp, dout, bs):
    n, din = x.shape
    d = hp["l1"]["W"].shape[1]
    dp = 128
    w2 = jnp.pad(hp["l2"]["W"], ((0, 0), (0, dp - dout)))
    b2 = jnp.pad(hp["l2"]["b"], (0, dp - dout)).reshape(1, -1)
    out = pl.pallas_call(
        _head_body,
        grid=(n // bs,),
        in_specs=[
            _row_block(bs, din),
            _full_block((din, d)), _full_block((1, d)),
            _full_block((d, dp)), _full_block((1, dp)),
        ],
        out_specs=_row_block(bs, dp),
        out_shape=jax.ShapeDtypeStruct((n, dp), jnp.float32),
    )(x, hp["l1"]["W"], hp["l1"]["b"].reshape(1, -1), w2, b2)
    return out[:, :dout]


_CH = 128


def _gather_rows(table, idx):
    e_rows = idx.shape[0]
    d = table.shape[1]
    per_w = (e_rows // _NW) & ~7
    rem = e_rows - per_w * _NW
    n_ch = per_w // _CH
    tail = per_w - n_ch * _CH
    mesh = plsc.VectorSubcoreMesh(core_axis_name="c", subcore_axis_name="s")

    @functools.partial(
        pl.kernel, mesh=mesh,
        out_type=jax.ShapeDtypeStruct((e_rows, d), table.dtype),
        scratch_types=[
            pltpu.VMEM((2, _CH), jnp.int32),
            pltpu.VMEM((2, _CH, d), table.dtype),
            pltpu.SemaphoreType.DMA,
        ],
    )
    def k(table_hbm, idx_hbm, out_hbm, idx_v, rows_v, gsem):
        wid = lax.axis_index("s") * _NC + lax.axis_index("c")
        base = wid * per_w

        def fire(g, slot):
            pltpu.sync_copy(idx_hbm.at[pl.ds(base + g * _CH, _CH)],
                            idx_v.at[slot])
            pltpu.async_copy(table_hbm.at[idx_v.at[slot]], rows_v.at[slot],
                             gsem)

        def body(g, carry):
            slot = lax.rem(g, 2)
            nslot = 1 - slot

            @pl.when(g + 1 < n_ch)
            def _():
                fire(g + 1, nslot)

            pltpu.make_async_copy(table_hbm.at[idx_v.at[slot]],
                                  rows_v.at[slot], gsem).wait()
            pltpu.sync_copy(rows_v.at[slot],
                            out_hbm.at[pl.ds(base + g * _CH, _CH)])
            return carry

        fire(0, 0)
        lax.fori_loop(0, n_ch, body, 0)

        def drain(dbase, length):
            pltpu.sync_copy(idx_hbm.at[pl.ds(dbase, length)],
                            idx_v.at[0, pl.ds(0, length)])
            pltpu.async_copy(table_hbm.at[idx_v.at[0, pl.ds(0, length)]],
                             rows_v.at[0, pl.ds(0, length)], gsem).wait()
            pltpu.sync_copy(rows_v.at[0, pl.ds(0, length)],
                            out_hbm.at[pl.ds(dbase, length)])

        if tail:
            drain(base + n_ch * _CH, tail)
        if rem:
            @pl.when(wid == 0)
            def _():
                rbase = _NW * per_w
                left = rem
                while left > 0:
                    ln = min(_CH, left)
                    drain(rbase, ln)
                    rbase += ln
                    left -= ln

    return k(table, idx)


def _gather_rows_bf16(table, idx):
    n, d = table.shape
    t32 = lax.bitcast_convert_type(table.reshape(n, d // 2, 2), jnp.int32)
    g = _gather_rows(t32, idx)
    return lax.bitcast_convert_type(g, jnp.bfloat16).reshape(idx.shape[0], d)


def _segment_sum(values, idx, n, init):
    e_rows, d = values.shape
    dh = d // _NC
    per_t = e_rows // _NS
    n_ch = per_t // _CH
    tail = per_t - n_ch * _CH
    rows_per_t = (n // _NS) & ~7
    rows_rem = n - _NS * rows_per_t
    mesh = plsc.VectorSubcoreMesh(core_axis_name="c", subcore_axis_name="s")

    @functools.partial(
        pl.kernel, mesh=mesh,
        out_type=jax.ShapeDtypeStruct((n, d), jnp.float32),
        scratch_types=[
            pltpu.VMEM_SHARED((n, dh), jnp.float32),
            pltpu.VMEM((2, _CH), jnp.int32),
            pltpu.VMEM((2, _CH, dh), jnp.float32),
            pltpu.VMEM((1, max(tail, 8)), jnp.int32),
            pltpu.SemaphoreType.DMA,
        ],
    )
    def k(init_hbm, idx_hbm, val_hbm, out_hbm, acc_sh, idx_v, val_v,
          tidx_v, vsem):
        cid = lax.axis_index("c")
        tid = lax.axis_index("s")
        base = tid * per_t
        col0 = cid * dh

        pltpu.sync_copy(
            init_hbm.at[pl.ds(tid * rows_per_t, rows_per_t),
                        pl.ds(col0, dh)],
            acc_sh.at[pl.ds(tid * rows_per_t, rows_per_t)])
        if rows_rem:
            @pl.when(tid == _NS - 1)
            def _():
                r0 = _NS * rows_per_t
                pltpu.sync_copy(
                    init_hbm.at[pl.ds(r0, rows_rem), pl.ds(col0, dh)],
                    acc_sh.at[pl.ds(r0, rows_rem)])
        plsc.subcore_barrier()

        def fire(g, slot):
            pltpu.sync_copy(idx_hbm.at[pl.ds(base + g * _CH, _CH)],
                            idx_v.at[slot])
            pltpu.async_copy(
                val_hbm.at[pl.ds(base + g * _CH, _CH), pl.ds(col0, dh)],
                val_v.at[slot], vsem)

        def body(g, carry):
            slot = lax.rem(g, 2)
            nslot = 1 - slot

            @pl.when(g + 1 < n_ch)
            def _():
                fire(g + 1, nslot)

            pltpu.make_async_copy(
                val_hbm.at[pl.ds(base + g * _CH, _CH), pl.ds(col0, dh)],
                val_v.at[slot], vsem).wait()
            pltpu.sync_copy(val_v.at[slot], acc_sh.at[idx_v.at[slot]],
                            add=True)
            return carry

        fire(0, 0)
        lax.fori_loop(0, n_ch, body, 0)
        if tail:
            tbase = base + n_ch * _CH
            pltpu.sync_copy(idx_hbm.at[pl.ds(tbase, tail)], tidx_v.at[0])
            pltpu.sync_copy(val_hbm.at[pl.ds(tbase, tail), pl.ds(col0, dh)],
                            val_v.at[0, pl.ds(0, tail)])
            pltpu.sync_copy(val_v.at[0, pl.ds(0, tail)],
                            acc_sh.at[tidx_v.at[0]], add=True)

        plsc.subcore_barrier()
        pltpu.sync_copy(
            acc_sh.at[pl.ds(tid * rows_per_t, rows_per_t)],
            out_hbm.at[pl.ds(tid * rows_per_t, rows_per_t), pl.ds(col0, dh)])
        if rows_rem:
            @pl.when(tid == _NS - 1)
            def _():
                r0 = _NS * rows_per_t
                pltpu.sync_copy(
                    acc_sh.at[pl.ds(r0, rows_rem)],
                    out_hbm.at[pl.ds(r0, rows_rem), pl.ds(col0, dh)])

    return k(init, idx, values)


def kernel(node_feats, edge_feats, random_feats, params, edge_index):
    src = edge_index[0]
    dst = edge_index[1]
    n = node_feats.shape[0]
    n_edges = edge_feats.shape[0]
    bs_n = 2000
    bs_e = 2000

    h = _node_embed(node_feats, random_feats,
                    params["scalar_emb"], params["rand_emb"], bs_n)
    e = _mlp_ln(edge_feats, params["edge_emb"], bs_e)

    conv_idx = 0
    d = N_HIDDEN
    ep = params["edge_upd"]
    eh = n_edges // 2
    src1, src2 = src[:eh], src[eh:]
    dst1, dst2 = dst[:eh], dst[eh:]
    zeros = jnp.zeros((n, d), jnp.float32)
    for _update in range(2):
        for _c in range(2):
            cp = params["convs"][conv_idx]
            conv_idx += 1
            hw = _matmul(h, cp["msg1"]["W"][:d], bs_n, jnp.bfloat16)
            g1 = _gather_rows_bf16(hw, src1)
            g2 = _gather_rows_bf16(hw, src2)
            m1 = _msg_mlp(g1, e, cp, bs_e, 0)
            m2 = _msg_mlp(g2, e, cp, bs_e, eh)
            a1 = _segment_sum(m1, dst1, n, zeros)
            a2 = _segment_sum(m2, dst2, n, zeros)
            h = _node_update(h, a1, a2, cp, bs_n)
        ha = _matmul(h, ep["l1"]["W"][:d], bs_n)
        hb = _matmul(h, ep["l1"]["W"][d:2 * d], bs_n)
        ga = _gather_rows(ha, src)
        gb = _gather_rows(hb, dst)
        e = _edge_update(ga, gb, e, ep, bs_e)

    a_logits = _head(h, params["head_a"], 16, bs_n)
    c_logits = _head(h, params["head_c"], 6, bs_n)
    e_logits = _head(e, params["head_e"], 5, bs_e)
    return (a_logits, c_logits, e_logits)


# speedup vs baseline: 2.4396x; 2.4396x over previous
import functools
import math

import jax
import jax.numpy as jnp
from jax import lax
from jax.experimental import pallas as pl
from jax.experimental.pallas import tpu as pltpu
from jax.experimental.pallas import tpu_sc as plsc

_NC = 2
_NS = 16
_NW = _NC * _NS

N_HIDDEN = 256
N_EDGE_HIDDEN = 128
MSG_NORM = 100.0
_EPS = 1e-5


def _silu(x):
    return x * (1.0 / (1.0 + jnp.exp(-x)))


def _layernorm(x, g, b):
    m = jnp.mean(x, axis=-1, keepdims=True)
    v = jnp.mean((x - m) ** 2, axis=-1, keepdims=True)
    return (x - m) * jax.lax.rsqrt(v + _EPS) * g + b


def _row_block(bs, d):
    return pl.BlockSpec((bs, d), lambda i: (i, 0))


def _full_block(shape):
    nd = len(shape)
    return pl.BlockSpec(shape, lambda i: (0,) * nd)


def _node_embed_body(x_ref, r_ref, w1_ref, b1_ref, w2_ref, b2_ref,
                     lng_ref, lnb_ref, rw1_ref, rb1_ref, rw2_ref, rb2_ref,
                     gate_ref, o_ref):
    x = x_ref[...]
    h = _silu(jnp.dot(x, w1_ref[...]) + b1_ref[...])
    h = _silu(jnp.dot(h, w2_ref[...]) + b2_ref[...])
    h = _layernorm(h, lng_ref[...], lnb_ref[...])
    r = _silu(jnp.dot(r_ref[...], rw1_ref[...]) + rb1_ref[...])
    r = jnp.dot(r, rw2_ref[...]) + rb2_ref[...]
    o_ref[...] = h + gate_ref[0, 0] * r


def _node_embed(x, r, se, re, bs):
    n, din = x.shape
    dr = r.shape[1]
    d = N_HIDDEN
    return pl.pallas_call(
        _node_embed_body,
        grid=(n // bs,),
        in_specs=[
            _row_block(bs, din), _row_block(bs, dr),
            _full_block((din, d)), _full_block((1, d)),
            _full_block((d, d)), _full_block((1, d)),
            _full_block((1, d)), _full_block((1, d)),
            _full_block((dr, d)), _full_block((1, d)),
            _full_block((d, d)), _full_block((1, d)),
            _full_block((1, 1)),
        ],
        out_specs=_row_block(bs, d),
        out_shape=jax.ShapeDtypeStruct((n, d), jnp.float32),
    )(x, r,
      se["l1"]["W"], se["l1"]["b"].reshape(1, -1),
      se["l2"]["W"], se["l2"]["b"].reshape(1, -1),
      se["ln"]["g"].reshape(1, -1), se["ln"]["b"].reshape(1, -1),
      re["l1"]["W"], re["l1"]["b"].reshape(1, -1),
      re["l2"]["W"], re["l2"]["b"].reshape(1, -1),
      re["gate"].reshape(1, 1))


def _mlp_ln_body(x_ref, w1_ref, b1_ref, w2_ref, b2_ref, lng_ref, lnb_ref,
                 o_ref):
    h = _silu(jnp.dot(x_ref[...], w1_ref[...]) + b1_ref[...])
    h = _silu(jnp.dot(h, w2_ref[...]) + b2_ref[...])
    o_ref[...] = _layernorm(h, lng_ref[...], lnb_ref[...])


def _mlp_ln(x, p, bs):
    n, din = x.shape
    d = p["l1"]["W"].shape[1]
    return pl.pallas_call(
        _mlp_ln_body,
        grid=(n // bs,),
        in_specs=[
            _row_block(bs, din),
            _full_block((din, d)), _full_block((1, d)),
            _full_block((d, d)), _full_block((1, d)),
            _full_block((1, d)), _full_block((1, d)),
        ],
        out_specs=_row_block(bs, d),
        out_shape=jax.ShapeDtypeStruct((n, d), jnp.float32),
    )(x, p["l1"]["W"], p["l1"]["b"].reshape(1, -1),
      p["l2"]["W"], p["l2"]["b"].reshape(1, -1),
      p["ln"]["g"].reshape(1, -1), p["ln"]["b"].reshape(1, -1))


def _matmul_body(x_ref, w_ref, o_ref):
    o_ref[...] = jnp.dot(x_ref[...], w_ref[...]).astype(o_ref.dtype)


def _matmul(x, w, bs):
    n, din = x.shape
    dout = w.shape[1]
    return pl.pallas_call(
        _matmul_body,
        grid=(n // bs,),
        in_specs=[_row_block(bs, din), _full_block((din, dout))],
        out_specs=_row_block(bs, dout),
        out_shape=jax.ShapeDtypeStruct((n, dout), jnp.float32),
    )(x, w)


def _matmul_pack_body(x_ref, w_ref, o_ref):
    y = jnp.dot(x_ref[...], w_ref[...])
    yb = lax.bitcast_convert_type(
        y.astype(jnp.bfloat16).astype(jnp.float32), jnp.uint32)
    half = y.shape[1] // 2
    lo = jnp.right_shift(yb[:, :half], 16)
    hi = jnp.bitwise_and(yb[:, half:], jnp.uint32(0xFFFF0000))
    o_ref[...] = lax.bitcast_convert_type(jnp.bitwise_or(lo, hi), jnp.int32)


def _matmul_pack(x, w, bs):
    n, din = x.shape
    dout = w.shape[1]
    return pl.pallas_call(
        _matmul_pack_body,
        grid=(n // bs,),
        in_specs=[_row_block(bs, din), _full_block((din, dout))],
        out_specs=_row_block(bs, dout // 2),
        out_shape=jax.ShapeDtypeStruct((n, dout // 2), jnp.int32),
    )(x, w)


def _unpack_bf16_pair(w32):
    w = lax.bitcast_convert_type(w32, jnp.uint32)
    lo = lax.bitcast_convert_type(jnp.left_shift(w, 16), jnp.float32)
    hi = lax.bitcast_convert_type(
        jnp.bitwise_and(w, jnp.uint32(0xFFFF0000)), jnp.float32)
    return jnp.concatenate([lo, hi], axis=1)


def _msg_body(hsw_ref, e_ref, w1b_ref, b1_ref, w2_ref, b2_ref, o_ref):
    h = jnp.dot(e_ref[...], w1b_ref[...])
    h = _silu(h + _unpack_bf16_pair(hsw_ref[...]) + b1_ref[...])
    o_ref[...] = _silu(jnp.dot(h, w2_ref[...]) + b2_ref[...])


def _msg_mlp(hsw, e, cp, bs, e_off=0):
    n = hsw.shape[0]
    d = N_HIDDEN
    de = N_EDGE_HIDDEN
    ob = e_off // bs
    w1 = cp["msg1"]["W"]
    return pl.pallas_call(
        _msg_body,
        grid=(n // bs,),
        in_specs=[
            _row_block(bs, d // 2),
            pl.BlockSpec((bs, de), lambda i: (i + ob, 0)),
            _full_block((de, d)), _full_block((1, d)),
            _full_block((d, d)), _full_block((1, d)),
        ],
        out_specs=_row_block(bs, d),
        out_shape=jax.ShapeDtypeStruct((n, d), jnp.float32),
    )(hsw, e, w1[d:], cp["msg1"]["b"].reshape(1, -1),
      cp["msg2"]["W"], cp["msg2"]["b"].reshape(1, -1))


def _node_update_body(h_ref, agg_ref, agg2_ref, ln1g_ref, ln1b_ref, w1_ref,
                      b1_ref, w2_ref, b2_ref, ln2g_ref, ln2b_ref, o_ref):
    agg = agg_ref[...] + agg2_ref[...]
    h = _layernorm(h_ref[...] + agg * (1.0 / MSG_NORM),
                   ln1g_ref[...], ln1b_ref[...])
    r = _silu(jnp.dot(h, w1_ref[...]) + b1_ref[...])
    r = _silu(jnp.dot(r, w2_ref[...]) + b2_ref[...])
    o_ref[...] = _layernorm(h + r, ln2g_ref[...], ln2b_ref[...])


def _node_update(h, agg, agg2, cp, bs):
    n = h.shape[0]
    d = N_HIDDEN
    return pl.pallas_call(
        _node_update_body,
        grid=(n // bs,),
        in_specs=[
            _row_block(bs, d), _row_block(bs, d), _row_block(bs, d),
            _full_block((1, d)), _full_block((1, d)),
            _full_block((d, d)), _full_block((1, d)),
            _full_block((d, d)), _full_block((1, d)),
            _full_block((1, d)), _full_block((1, d)),
        ],
        out_specs=_row_block(bs, d),
        out_shape=jax.ShapeDtypeStruct((n, d), jnp.float32),
    )(h, agg, agg2,
      cp["ln1"]["g"].reshape(1, -1), cp["ln1"]["b"].reshape(1, -1),
      cp["upd1"]["W"], cp["upd1"]["b"].reshape(1, -1),
      cp["upd2"]["W"], cp["upd2"]["b"].reshape(1, -1),
      cp["ln2"]["g"].reshape(1, -1), cp["ln2"]["b"].reshape(1, -1))


def _edge_update_body(ga_ref, gb_ref, e_ref, w1c_ref, b1_ref, w2_ref,
                      b2_ref, lng_ref, lnb_ref, o_ref):
    h = jnp.dot(e_ref[...], w1c_ref[...])
    h = _silu(h + ga_ref[...].astype(jnp.float32)
              + gb_ref[...].astype(jnp.float32) + b1_ref[...])
    eo = _silu(jnp.dot(h, w2_ref[...]) + b2_ref[...])
    o_ref[...] = _layernorm(e_ref[...] + eo, lng_ref[...], lnb_ref[...])


def _edge_update(ga, gb, e, ep, bs):
    n = ga.shape[0]
    de = N_EDGE_HIDDEN
    w1 = ep["l1"]["W"]
    d = N_HIDDEN
    return pl.pallas_call(
        _edge_update_body,
        grid=(n // bs,),
        in_specs=[
            _row_block(bs, de), _row_block(bs, de), _row_block(bs, de),
            _full_block((de, de)), _full_block((1, de)),
            _full_block((de, de)), _full_block((1, de)),
            _full_block((1, de)), _full_block((1, de)),
        ],
        out_specs=_row_block(bs, de),
        out_shape=jax.ShapeDtypeStruct((n, de), jnp.float32),
    )(ga, gb, e, w1[2 * d:],
      ep["l1"]["b"].reshape(1, -1),
      ep["l2"]["W"], ep["l2"]["b"].reshape(1, -1),
      ep["ln"]["g"].reshape(1, -1), ep["ln"]["b"].reshape(1, -1))


def _head_body(x_ref, w1_ref, b1_ref, w2_ref, b2_ref, o_ref):
    h = _silu(jnp.dot(x_ref[...], w1_ref[...]) + b1_ref[...])
    o_ref[...] = jnp.dot(h, w2_ref[...]) + b2_ref[...]


def _head(x, hp, dout, bs):
    n, din = x.shape
    d = hp["l1"]["W"].shape[1]
    dp = 128
    w2 = jnp.pad(hp["l2"]["W"], ((0, 0), (0, dp - dout)))
    b2 = jnp.pad(hp["l2"]["b"], (0, dp - dout)).reshape(1, -1)
    out = pl.pallas_call(
        _head_body,
        grid=(n // bs,),
        in_specs=[
            _row_block(bs, din),
            _full_block((din, d)), _full_block((1, d)),
            _full_block((d, dp)), _full_block((1, dp)),
        ],
        out_specs=_row_block(bs, dp),
        out_shape=jax.ShapeDtypeStruct((n, dp), jnp.float32),
    )(x, hp["l1"]["W"], hp["l1"]["b"].reshape(1, -1), w2, b2)
    return out[:, :dout]


_CH = 128


def _gather_rows(table, idx):
    e_rows = idx.shape[0]
    d = table.shape[1]
    per_w = (e_rows // _NW) & ~7
    rem = e_rows - per_w * _NW
    n_ch = per_w // _CH
    tail = per_w - n_ch * _CH
    mesh = plsc.VectorSubcoreMesh(core_axis_name="c", subcore_axis_name="s")

    @functools.partial(
        pl.kernel, mesh=mesh,
        out_type=jax.ShapeDtypeStruct((e_rows, d), table.dtype),
        scratch_types=[
            pltpu.VMEM((2, _CH), jnp.int32),
            pltpu.VMEM((2, _CH, d), table.dtype),
            pltpu.SemaphoreType.DMA,
        ],
    )
    def k(table_hbm, idx_hbm, out_hbm, idx_v, rows_v, gsem):
        wid = lax.axis_index("s") * _NC + lax.axis_index("c")
        base = wid * per_w

        def fire(g, slot):
            pltpu.sync_copy(idx_hbm.at[pl.ds(base + g * _CH, _CH)],
                            idx_v.at[slot])
            pltpu.async_copy(table_hbm.at[idx_v.at[slot]], rows_v.at[slot],
                             gsem)

        def body(g, carry):
            slot = lax.rem(g, 2)
            nslot = 1 - slot

            @pl.when(g + 1 < n_ch)
            def _():
                fire(g + 1, nslot)

            pltpu.make_async_copy(table_hbm.at[idx_v.at[slot]],
                                  rows_v.at[slot], gsem).wait()
            pltpu.sync_copy(rows_v.at[slot],
                            out_hbm.at[pl.ds(base + g * _CH, _CH)])
            return carry

        fire(0, 0)
        lax.fori_loop(0, n_ch, body, 0)

        def drain(dbase, length):
            pltpu.sync_copy(idx_hbm.at[pl.ds(dbase, length)],
                            idx_v.at[0, pl.ds(0, length)])
            pltpu.async_copy(table_hbm.at[idx_v.at[0, pl.ds(0, length)]],
                             rows_v.at[0, pl.ds(0, length)], gsem).wait()
            pltpu.sync_copy(rows_v.at[0, pl.ds(0, length)],
                            out_hbm.at[pl.ds(dbase, length)])

        if tail:
            drain(base + n_ch * _CH, tail)
        if rem:
            @pl.when(wid == 0)
            def _():
                rbase = _NW * per_w
                left = rem
                while left > 0:
                    ln = min(_CH, left)
                    drain(rbase, ln)
                    rbase += ln
                    left -= ln

    return k(table, idx)


def _segment_sum(values, idx, n, init):
    e_rows, d = values.shape
    dh = d // _NC
    per_t = e_rows // _NS
    n_ch = per_t // _CH
    tail = per_t - n_ch * _CH
    rows_per_t = (n // _NS) & ~7
    rows_rem = n - _NS * rows_per_t
    mesh = plsc.VectorSubcoreMesh(core_axis_name="c", subcore_axis_name="s")

    @functools.partial(
        pl.kernel, mesh=mesh,
        out_type=jax.ShapeDtypeStruct((n, d), jnp.float32),
        scratch_types=[
            pltpu.VMEM_SHARED((n, dh), jnp.float32),
            pltpu.VMEM((2, _CH), jnp.int32),
            pltpu.VMEM((2, _CH, dh), jnp.float32),
            pltpu.VMEM((1, max(tail, 8)), jnp.int32),
            pltpu.SemaphoreType.DMA,
        ],
    )
    def k(init_hbm, idx_hbm, val_hbm, out_hbm, acc_sh, idx_v, val_v,
          tidx_v, vsem):
        cid = lax.axis_index("c")
        tid = lax.axis_index("s")
        base = tid * per_t
        col0 = cid * dh

        pltpu.sync_copy(
            init_hbm.at[pl.ds(tid * rows_per_t, rows_per_t),
                        pl.ds(col0, dh)],
            acc_sh.at[pl.ds(tid * rows_per_t, rows_per_t)])
        if rows_rem:
            @pl.when(tid == _NS - 1)
            def _():
                r0 = _NS * rows_per_t
                pltpu.sync_copy(
                    init_hbm.at[pl.ds(r0, rows_rem), pl.ds(col0, dh)],
                    acc_sh.at[pl.ds(r0, rows_rem)])
        plsc.subcore_barrier()

        def fire(g, slot):
            pltpu.sync_copy(idx_hbm.at[pl.ds(base + g * _CH, _CH)],
                            idx_v.at[slot])
            pltpu.async_copy(
                val_hbm.at[pl.ds(base + g * _CH, _CH), pl.ds(col0, dh)],
                val_v.at[slot], vsem)

        def body(g, carry):
            slot = lax.rem(g, 2)
            nslot = 1 - slot

            @pl.when(g + 1 < n_ch)
            def _():
                fire(g + 1, nslot)

            pltpu.make_async_copy(
                val_hbm.at[pl.ds(base + g * _CH, _CH), pl.ds(col0, dh)],
                val_v.at[slot], vsem).wait()
            pltpu.sync_copy(val_v.at[slot], acc_sh.at[idx_v.at[slot]],
                            add=True)
            return carry

        fire(0, 0)
        lax.fori_loop(0, n_ch, body, 0)
        if tail:
            tbase = base + n_ch * _CH
            pltpu.sync_copy(idx_hbm.at[pl.ds(tbase, tail)], tidx_v.at[0])
            pltpu.sync_copy(val_hbm.at[pl.ds(tbase, tail), pl.ds(col0, dh)],
                            val_v.at[0, pl.ds(0, tail)])
            pltpu.sync_copy(val_v.at[0, pl.ds(0, tail)],
                            acc_sh.at[tidx_v.at[0]], add=True)

        plsc.subcore_barrier()
        pltpu.sync_copy(
            acc_sh.at[pl.ds(tid * rows_per_t, rows_per_t)],
            out_hbm.at[pl.ds(tid * rows_per_t, rows_per_t), pl.ds(col0, dh)])
        if rows_rem:
            @pl.when(tid == _NS - 1)
            def _():
                r0 = _NS * rows_per_t
                pltpu.sync_copy(
                    acc_sh.at[pl.ds(r0, rows_rem)],
                    out_hbm.at[pl.ds(r0, rows_rem), pl.ds(col0, dh)])

    return k(init, idx, values)


def kernel(node_feats, edge_feats, random_feats, params, edge_index):
    src = edge_index[0]
    dst = edge_index[1]
    n = node_feats.shape[0]
    n_edges = edge_feats.shape[0]
    bs_n = 2000
    bs_e = 2000

    h = _node_embed(node_feats, random_feats,
                    params["scalar_emb"], params["rand_emb"], bs_n)
    e = _mlp_ln(edge_feats, params["edge_emb"], bs_e)

    conv_idx = 0
    d = N_HIDDEN
    ep = params["edge_upd"]
    eh = n_edges // 2
    src1, src2 = src[:eh], src[eh:]
    dst1, dst2 = dst[:eh], dst[eh:]
    zeros = jnp.zeros((n, d), jnp.float32)
    for _update in range(2):
        for _c in range(2):
            cp = params["convs"][conv_idx]
            conv_idx += 1
            hw = _matmul_pack(h, cp["msg1"]["W"][:d], bs_n)
            g1 = _gather_rows(hw, src1)
            g2 = _gather_rows(hw, src2)
            m1 = _msg_mlp(g1, e, cp, bs_e, 0)
            m2 = _msg_mlp(g2, e, cp, bs_e, eh)
            a1 = _segment_sum(m1, dst1, n, zeros)
            a2 = _segment_sum(m2, dst2, n, zeros)
            h = _node_update(h, a1, a2, cp, bs_n)
        ha = _matmul(h, ep["l1"]["W"][:d], bs_n)
        hb = _matmul(h, ep["l1"]["W"][d:2 * d], bs_n)
        ga = _gather_rows(ha, src)
        gb = _gather_rows(hb, dst)
        e = _edge_update(ga, gb, e, ep, bs_e)

    a_logits = _head(h, params["head_a"], 16, bs_n)
    c_logits = _head(h, params["head_c"], 6, bs_n)
    e_logits = _head(e, params["head_e"], 5, bs_e)
    return (a_logits, c_logits, e_logits)
